# Initial kernel scaffold; baseline (speedup 1.0000x reference)
import jax
import jax.numpy as jnp
from jax.experimental import pallas as pl


def kernel(features_per_image, lines_per_im):
    raise NotImplementedError("write your pallas kernel here")


# speedup vs baseline: 16.1690x; 16.1690x over previous
import jax
import jax.numpy as jnp
import numpy as np
from jax import lax
from jax.experimental import pallas as pl
from jax.experimental.pallas import tpu as pltpu
from jax.experimental.pallas import tpu_sc as plsc

N_SAMP = 32
N_FINAL = 8
FEAT = 128
H = 128
W = 128
L_LINES = 8192
NW = 32
LPW = L_LINES // NW


def _sc_body(table_hbm, lines_hbm, t_hbm, out_hbm,
             lines_v, t_v, idx_v, rows_v, out_v, sem, osem):
    wid = lax.axis_index("c") * 16 + lax.axis_index("s")
    base = wid * LPW
    pltpu.sync_copy(lines_hbm.at[pl.ds(base * 4, LPW * 4)],
                    lines_v.at[pl.ds(0, LPW * 4)])
    pltpu.sync_copy(t_hbm, t_v)


    def line_body(i, carry):
        ln = lines_v[pl.ds(4 * i, 16)]
        ux = ln[0]
        uy = ln[1]
        vx = ln[2]
        vy = ln[3]
        wregs = []
        for h in range(2):
            t = t_v[pl.ds(16 * h, 16)]
            omt = 1.0 - t
            px = ux * t + vx * omt - 0.5
            py = uy * t + vy * omt - 0.5
            x0i = jnp.clip(px.astype(jnp.int32), 0, W - 1)
            y0i = jnp.clip(py.astype(jnp.int32), 0, H - 1)
            x1i = jnp.minimum(x0i + 1, W - 1)
            y1i = jnp.minimum(y0i + 1, H - 1)
            x0 = x0i.astype(jnp.float32)
            y0 = y0i.astype(jnp.float32)
            x1 = x1i.astype(jnp.float32)
            y1 = y1i.astype(jnp.float32)
            wx0 = x1 - px
            wx1 = px - x0
            wy0 = y1 - py
            wy1 = py - y0
            wregs.append((wy0 * wx0, wy0 * wx1, wy1 * wx0, wy1 * wx1))
            yb0 = y0i * W
            yb1 = y1i * W
            idx_v[pl.ds(16 * h, 16)] = yb0 + x0i
            idx_v[pl.ds(32 + 16 * h, 16)] = yb0 + x1i
            idx_v[pl.ds(64 + 16 * h, 16)] = yb1 + x0i
            idx_v[pl.ds(96 + 16 * h, 16)] = yb1 + x1i

        pltpu.async_copy(table_hbm.at[idx_v], rows_v, sem).wait()

        for f in range(N_FINAL):
            acc = [None] * 8
            for j in range(4):
                s = 4 * f + j
                wr = wregs[s // 16]
                e = s % 16
                w00 = jnp.full((16,), wr[0][e])
                w01 = jnp.full((16,), wr[1][e])
                w10 = jnp.full((16,), wr[2][e])
                w11 = jnp.full((16,), wr[3][e])
                for g in range(8):
                    cs = pl.ds(16 * g, 16)
                    v = (rows_v[s, cs] * w00 + rows_v[32 + s, cs] * w01
                         + rows_v[64 + s, cs] * w10 + rows_v[96 + s, cs] * w11)
                    acc[g] = v if j == 0 else jnp.maximum(acc[g], v)
            for g in range(8):
                out_v[pl.ds(128 * f + 16 * g, 16)] = acc[g]

        pltpu.sync_copy(out_v, out_hbm.at[base + i])
        return carry

    lax.fori_loop(0, LPW, line_body, 0)


def kernel(features_per_image, lines_per_im):
    table = jnp.transpose(features_per_image, (1, 2, 0)).reshape(H * W, FEAT)
    lines_flat = lines_per_im.reshape(L_LINES * 4)
    tspan = jnp.asarray(np.linspace(0.0, 1.0, N_SAMP, dtype=np.float32))
    mesh = plsc.VectorSubcoreMesh(core_axis_name="c", subcore_axis_name="s")
    sc = pl.kernel(
        _sc_body,
        out_type=jax.ShapeDtypeStruct((L_LINES, N_FINAL * FEAT), jnp.float32),
        mesh=mesh,
        scratch_types=[
            pltpu.VMEM((LPW * 4 + 16,), jnp.float32),
            pltpu.VMEM((N_SAMP,), jnp.float32),
            pltpu.VMEM((4 * N_SAMP,), jnp.int32),
            pltpu.VMEM((4 * N_SAMP, FEAT), jnp.float32),
            pltpu.VMEM((N_FINAL * FEAT,), jnp.float32),
            pltpu.SemaphoreType.DMA,
            pltpu.SemaphoreType.DMA,
        ],
    )
    out = sc(table, lines_flat, tspan)
    return jnp.transpose(out.reshape(L_LINES, N_FINAL, FEAT),
                         (0, 2, 1)).reshape(L_LINES, N_FINAL * FEAT)
